# Initial kernel scaffold; baseline (speedup 1.0000x reference)
import jax
import jax.numpy as jnp
from jax.experimental import pallas as pl


def kernel(points_sdf, W1, b1, W2, b2, bias):
    raise NotImplementedError("write your pallas kernel here")


# speedup vs baseline: 5.0022x; 5.0022x over previous
import functools

import jax
import jax.numpy as jnp
from jax.experimental import pallas as pl

EDGE_LENGTH = 0.01
FILTER_K = 32
CENTER_N = 4096
OUT_CH = 32
HIDDEN = 16

CB = 256


def _fc_kernel(pts_ref, ctr_ref, W1_ref, b1_ref, W2_ref, b2_ref, bias_ref,
               out_ref, *, n_points, k_sel):
    pts = pts_ref[0]
    coords = pts[:, :3]
    centers = ctr_ref[0]
    ccoords = centers[:, :3]

    c2 = jnp.sum(ccoords * ccoords, axis=1, keepdims=True)
    p2 = jnp.sum(coords * coords, axis=1)[None, :]
    dot = jax.lax.dot_general(
        ccoords, coords, (((1,), (1,)), ((), ())),
        preferred_element_type=jnp.float32)
    d2 = c2 + p2 - 2.0 * dot

    iota = jax.lax.broadcasted_iota(jnp.int32, d2.shape, 1)
    W1 = W1_ref[...]
    b1 = b1_ref[0]
    W2 = W2_ref[...]
    b2 = b2_ref[0]

    def body(_, carry):
        m, i, acc = carry
        live = (d2 > m) | ((d2 == m) & (iota > i))
        dm = jnp.where(live, d2, jnp.inf)
        m2 = jnp.min(dm, axis=1, keepdims=True)
        i2 = jnp.min(jnp.where(dm == m2, iota, n_points),
                     axis=1, keepdims=True)
        onehot = (iota == i2).astype(jnp.float32)
        sel = jax.lax.dot_general(
            onehot, pts, (((1,), (0,)), ((), ())),
            preferred_element_type=jnp.float32)
        rel = (sel[:, :3] - ccoords) * (1.0 / EDGE_LENGTH)
        h = jax.nn.relu(
            jax.lax.dot_general(rel, W1, (((1,), (0,)), ((), ())),
                                preferred_element_type=jnp.float32) + b1)
        w = jax.lax.dot_general(
            h, W2, (((1,), (0,)), ((), ())),
            preferred_element_type=jnp.float32) + b2
        acc = acc + sel[:, 3:4] * w
        return m2, i2, acc

    m0 = jnp.full((centers.shape[0], 1), -jnp.inf, dtype=jnp.float32)
    i0 = jnp.full((centers.shape[0], 1), -1, dtype=jnp.int32)
    acc0 = jnp.zeros((centers.shape[0], OUT_CH), dtype=jnp.float32)
    _, _, acc = jax.lax.fori_loop(0, k_sel, body, (m0, i0, acc0))

    out_ref[0] = acc * (1.0 / k_sel) + bias_ref[0]


def kernel(points_sdf, W1, b1, W2, b2, bias):
    B, N, _ = points_sdf.shape
    nblk = CENTER_N // CB

    fn = functools.partial(_fc_kernel, n_points=N, k_sel=FILTER_K)
    out = pl.pallas_call(
        fn,
        grid=(B, nblk),
        in_specs=[
            pl.BlockSpec((1, N, 4), lambda b, j: (b, 0, 0)),
            pl.BlockSpec((1, CB, 4), lambda b, j: (b, j, 0)),
            pl.BlockSpec((3, HIDDEN), lambda b, j: (0, 0)),
            pl.BlockSpec((1, HIDDEN), lambda b, j: (0, 0)),
            pl.BlockSpec((HIDDEN, OUT_CH), lambda b, j: (0, 0)),
            pl.BlockSpec((1, OUT_CH), lambda b, j: (0, 0)),
            pl.BlockSpec((1, OUT_CH), lambda b, j: (0, 0)),
        ],
        out_specs=pl.BlockSpec((1, CB, OUT_CH), lambda b, j: (b, j, 0)),
        out_shape=jax.ShapeDtypeStruct((B, CENTER_N, OUT_CH), jnp.float32),
    )(points_sdf, points_sdf[:, :CENTER_N, :], W1, b1[None, :], W2,
      b2[None, :], bias[None, :])
    return out
